# Initial kernel scaffold; baseline (speedup 1.0000x reference)
import jax
import jax.numpy as jnp
from jax.experimental import pallas as pl


def kernel(x, embed0, embed1, embed2, embed3, thresholds):
    raise NotImplementedError("write your pallas kernel here")


# speedup vs baseline: 1.3756x; 1.3756x over previous
import jax
import jax.numpy as jnp
from jax.experimental import pallas as pl
from jax.experimental.pallas import tpu as pltpu

_B, _D, _T = 16, 256, 512
_K = 1024
_NCB = 4
_NTOK = _B * _T
_W = 2
_STEPS = _B // _W


def _rvq_body(thr_ref, x_ref,
              em0_ref, em1_ref, em2_ref, em3_ref,
              et0_ref, et1_ref, et2_ref, et3_ref,
              e20_ref, e21_ref, e22_ref, e23_ref,
              qt_ref, stats_ref, util_scr):
    b = pl.program_id(0)

    @pl.when(b == 0)
    def _init():
        for i in range(16):
            stats_ref[i] = jnp.float32(0.0)
        util_scr[...] = jnp.zeros_like(util_scr)

    xs = [x_ref[w] for w in range(_W)]

    fcf = []
    for xb in xs:
        energy = jnp.sum(xb * xb, axis=0, keepdims=True) * jnp.float32(1.0 / _D)
        e_min = jnp.min(energy, axis=1, keepdims=True)
        e_max = jnp.max(energy, axis=1, keepdims=True)
        e_rng = jnp.maximum(e_max - e_min, jnp.float32(1e-6))
        comp = (energy - e_min) / e_rng
        fcf.append(jnp.float32(1.0)
                   + (comp > thr_ref[0]).astype(jnp.float32)
                   + (comp > thr_ref[1]).astype(jnp.float32)
                   + (comp > thr_ref[2]).astype(jnp.float32))

    iota_k = jax.lax.broadcasted_iota(jnp.int32, (_K, _T), 0)
    em_refs = (em0_ref, em1_ref, em2_ref, em3_ref)
    et_refs = (et0_ref, et1_ref, et2_ref, et3_ref)
    e2_refs = (e20_ref, e21_ref, e22_ref, e23_ref)

    residual = list(xs)
    qtot = [jnp.zeros_like(xs[0]) for _ in range(_W)]
    for s in range(_NCB):
        em16 = em_refs[s][...]
        et = et_refs[s][...]
        e2 = e2_refs[s][...]
        x2 = [jnp.sum(r * r, axis=0, keepdims=True) for r in residual]
        r16 = [r.astype(jnp.bfloat16) for r in residual]
        dot = [jax.lax.dot_general(em16, r, (((1,), (0,)), ((), ())),
                                   preferred_element_type=jnp.float32)
               for r in r16]
        d2 = [(x2[w] + e2) - jnp.float32(2.0) * dot[w] for w in range(_W)]
        dist = [jnp.sqrt(jnp.maximum(d, jnp.float32(0.0))) for d in d2]
        m = [jnp.min(d, axis=0, keepdims=True) for d in dist]
        idx = [jnp.min(jnp.where(dist[w] == m[w], iota_k, _K), axis=0,
                       keepdims=True) for w in range(_W)]
        onehot = [(iota_k == i).astype(jnp.float32) for i in idx]
        used = onehot[0]
        for w in range(1, _W):
            used = jnp.maximum(used, onehot[w])
        util_scr[s] = jnp.maximum(util_scr[s],
                                  jnp.max(used, axis=1, keepdims=True))
        ihi = [i >> 7 for i in idx]
        ilo = [jnp.broadcast_to(i & 127, (_D, _T)) for i in idx]
        q = [jnp.zeros((_D, _T), jnp.float32) for _ in range(_W)]
        for j in range(_K // 128):
            etj = et[:, j * 128:(j + 1) * 128]
            for w in range(_W):
                qj = jnp.take_along_axis(etj, ilo[w], axis=1,
                                         mode="promise_in_bounds")
                q[w] = jnp.where(ihi[w] == j, qj, q[w])
        maskf = [(f > jnp.float32(s)).astype(jnp.float32) for f in fcf]
        diff = [residual[w] - q[w] for w in range(_W)]
        ssum = jnp.sum(diff[0] * diff[0])
        csum = jnp.sum(maskf[0])
        for w in range(1, _W):
            ssum = ssum + jnp.sum(diff[w] * diff[w])
            csum = csum + jnp.sum(maskf[w])
        stats_ref[s] = stats_ref[s] + ssum
        stats_ref[4 + s] = stats_ref[4 + s] + csum
        qst = [residual[w] + (q[w] - residual[w]) for w in range(_W)]
        qtot = [qtot[w] + qst[w] * maskf[w] for w in range(_W)]
        residual = [residual[w] - qst[w] * maskf[w] for w in range(_W)]

    for w in range(_W):
        qt_ref[w] = qtot[w]

    @pl.when(b == _STEPS - 1)
    def _finish():
        for s in range(_NCB):
            stats_ref[8 + s] = jnp.sum(util_scr[s])


def kernel(x, embed0, embed1, embed2, embed3, thresholds):
    embeds = (embed0, embed1, embed2, embed3)
    em16 = [e.astype(jnp.bfloat16) for e in embeds]
    ets = [e.T for e in embeds]
    e2s = [jnp.sum(e ** 2, axis=1)[:, None] for e in embeds]

    const_spec2 = lambda shape: pl.BlockSpec(shape, lambda b: (0, 0))
    qt, stats = pl.pallas_call(
        _rvq_body,
        grid=(_STEPS,),
        in_specs=[
            pl.BlockSpec(memory_space=pltpu.SMEM),
            pl.BlockSpec((_W, _D, _T), lambda b: (b, 0, 0)),
            *[const_spec2((_K, _D)) for _ in range(4)],
            *[const_spec2((_D, _K)) for _ in range(4)],
            *[const_spec2((_K, 1)) for _ in range(4)],
        ],
        out_specs=[
            pl.BlockSpec((_W, _D, _T), lambda b: (b, 0, 0)),
            pl.BlockSpec(memory_space=pltpu.SMEM),
        ],
        out_shape=[
            jax.ShapeDtypeStruct((_B, _D, _T), jnp.float32),
            jax.ShapeDtypeStruct((16,), jnp.float32),
        ],
        scratch_shapes=[pltpu.VMEM((_NCB, _K, 1), jnp.float32)],
    )(thresholds, x, *em16, *ets, *e2s)

    total_commit = jnp.float32(0.0)
    total_util = jnp.float32(0.0)
    total_bits = jnp.float32(0.0)
    for s in range(_NCB):
        c = stats[s] / jnp.float32(_NTOK * _D)
        commit_s = c + c
        total_commit = total_commit + commit_s * (stats[4 + s] / jnp.float32(_NTOK))
        total_util = total_util + stats[8 + s] / jnp.float32(_K)
        total_bits = total_bits + stats[4 + s]
    avg_util = total_util / jnp.float32(_NCB)
    avg_bits_per_frame = total_bits / jnp.float32(_NTOK)
    mean_frame_counts = total_bits / jnp.float32(_NTOK)
    return (qt, total_commit, avg_util, mean_frame_counts, avg_bits_per_frame)
